# baseline (speedup 1.0000x reference)
import jax
import jax.numpy as jnp
from jax import lax
from jax.experimental import pallas as pl
from jax.experimental.pallas import tpu as pltpu
from jax.experimental.pallas import tpu_sc as plsc

NC = 2
NS = 16
NW = NC * NS
L = 16

_B, _S, _D = 1024, 200, 64
_DP = 2 * _D
_ROWS_W = _B // NW


def _make_kernel():
    mesh = plsc.VectorSubcoreMesh(
        core_axis_name="c", subcore_axis_name="s",
        num_cores=NC, num_subcores=NS)

    @pl.kernel(
        out_type=jax.ShapeDtypeStruct((_B, _S, _D), jnp.float32),
        mesh=mesh,
        compiler_params=pltpu.CompilerParams(use_tc_tiling_on_sc=True),
        scratch_types=[
            pltpu.VMEM((2 * _S,), jnp.int32),
            pltpu.VMEM((_S, _D), jnp.float32),
            pltpu.VMEM((2, _S, _DP), jnp.float32),
            pltpu.VMEM((2, _S, _D), jnp.float32),
            pltpu.SemaphoreType.DMA,
            pltpu.SemaphoreType.DMA,
            pltpu.SemaphoreType.DMA,
        ],
    )
    def k(x_hbm, table_hbm, pe_hbm, out_hbm, idx_v, pe_v, rows_v, sum_v,
          isem, gsem, psem):
        wid = lax.axis_index("s") * NC + lax.axis_index("c")
        base = wid * _ROWS_W
        pltpu.sync_copy(pe_hbm.at[pl.ds(0, _S)], pe_v)

        def idx_copy(kk, p):
            pltpu.async_copy(
                x_hbm.at[pl.ds((base + kk) * _S, _S)],
                idx_v.at[pl.ds(p * _S, _S)], isem)

        def chunk_body(kk, carry):
            p = kk % 2
            pltpu.make_async_copy(
                table_hbm.at[idx_v.at[pl.ds(0, _S)]], rows_v.at[p], gsem).wait()

            @pl.when(kk + 1 < _ROWS_W)
            def _():
                pltpu.make_async_copy(
                    x_hbm.at[pl.ds(0, _S)], idx_v.at[pl.ds(0, _S)], isem).wait()
                pltpu.async_copy(
                    table_hbm.at[idx_v.at[pl.ds((1 - p) * _S, _S)]],
                    rows_v.at[1 - p], gsem)

            @pl.when(kk + 2 < _ROWS_W)
            def _():
                idx_copy(kk + 2, p)

            @pl.when(kk >= 2)
            def _():
                pltpu.make_async_copy(
                    sum_v.at[0], out_hbm.at[base], psem).wait()

            rp = rows_v.at[p]
            sp = sum_v.at[p]

            @plsc.parallel_loop(0, _S, unroll=16)
            def row_body(r):
                for c in range(_D // L):
                    sl = pl.ds(c * L, L)
                    sp[r, sl] = rp[r, sl] + pe_v[r, sl]
            pltpu.async_copy(sp, out_hbm.at[base + kk], psem)
            return carry

        pltpu.sync_copy(x_hbm.at[pl.ds(base * _S, _S)], idx_v.at[pl.ds(0, _S)])
        pltpu.async_copy(
            table_hbm.at[idx_v.at[pl.ds(0, _S)]], rows_v.at[0], gsem)
        idx_copy(1, 1)
        lax.fori_loop(0, _ROWS_W, chunk_body, 0)
        for _ in range(2):
            pltpu.make_async_copy(sum_v.at[0], out_hbm.at[base], psem).wait()

    return k


_kernel_call = _make_kernel()


def kernel(x, token_table, pe):
    tab128 = jnp.pad(token_table, ((0, 0), (0, _D)))
    return _kernel_call(x.reshape(-1), tab128, pe)


# speedup vs baseline: 1.0036x; 1.0036x over previous
import jax
import jax.numpy as jnp
from jax import lax
from jax.experimental import pallas as pl
from jax.experimental.pallas import tpu as pltpu
from jax.experimental.pallas import tpu_sc as plsc

NC = 2
NS = 16
NW = NC * NS
L = 16

_B, _S, _D = 1024, 200, 64
_DP = 2 * _D
_ROWS_W = _B // NW


def _make_kernel():
    mesh = plsc.VectorSubcoreMesh(
        core_axis_name="c", subcore_axis_name="s",
        num_cores=NC, num_subcores=NS)

    @pl.kernel(
        out_type=jax.ShapeDtypeStruct((_B, _S, _D), jnp.float32),
        mesh=mesh,
        compiler_params=pltpu.CompilerParams(use_tc_tiling_on_sc=True),
        scratch_types=[
            pltpu.VMEM((2 * _S,), jnp.int32),
            pltpu.VMEM((_S, _D), jnp.float32),
            pltpu.VMEM((2, _S, _DP), jnp.float32),
            pltpu.VMEM((2, _S, _D), jnp.float32),
            pltpu.SemaphoreType.DMA,
            pltpu.SemaphoreType.DMA,
            pltpu.SemaphoreType.DMA,
        ],
    )
    def k(x_hbm, table_hbm, pe_hbm, out_hbm, idx_v, pe_v, rows_v, sum_v,
          isem, gsem, psem):
        wid = lax.axis_index("s") * NC + lax.axis_index("c")
        base = wid * _ROWS_W
        pltpu.sync_copy(pe_hbm.at[pl.ds(0, _S)], pe_v)

        def idx_copy(kk, p):
            pltpu.async_copy(
                x_hbm.at[pl.ds((base + kk) * _S, _S)],
                idx_v.at[pl.ds(p * _S, _S)], isem)

        def chunk_body(kk, carry):
            p = kk % 2
            pltpu.make_async_copy(
                table_hbm.at[idx_v.at[pl.ds(0, _S)]], rows_v.at[p], gsem).wait()

            @pl.when(kk + 1 < _ROWS_W)
            def _():
                pltpu.make_async_copy(
                    x_hbm.at[pl.ds(0, _S)], idx_v.at[pl.ds(0, _S)], isem).wait()
                pltpu.async_copy(
                    table_hbm.at[idx_v.at[pl.ds((1 - p) * _S, _S)]],
                    rows_v.at[1 - p], gsem)

            @pl.when(kk + 2 < _ROWS_W)
            def _():
                idx_copy(kk + 2, p)

            @pl.when(kk >= 2)
            def _():
                pltpu.make_async_copy(
                    sum_v.at[0], out_hbm.at[base], psem).wait()

            rp = rows_v.at[p]
            sp = sum_v.at[p]

            @plsc.parallel_loop(0, _S, unroll=8)
            def row_body(r):
                for c in range(_D // L):
                    sl = pl.ds(c * L, L)
                    sp[r, sl] = rp[r, sl] + pe_v[r, sl]
            pltpu.async_copy(sp, out_hbm.at[base + kk], psem)
            return carry

        pltpu.sync_copy(x_hbm.at[pl.ds(base * _S, _S)], idx_v.at[pl.ds(0, _S)])
        pltpu.async_copy(
            table_hbm.at[idx_v.at[pl.ds(0, _S)]], rows_v.at[0], gsem)
        idx_copy(1, 1)
        lax.fori_loop(0, _ROWS_W, chunk_body, 0)
        for _ in range(2):
            pltpu.make_async_copy(sum_v.at[0], out_hbm.at[base], psem).wait()

    return k


_kernel_call = _make_kernel()


def kernel(x, token_table, pe):
    tab128 = jnp.pad(token_table, ((0, 0), (0, _D)))
    return _kernel_call(x.reshape(-1), tab128, pe)
